# Initial kernel scaffold; baseline (speedup 1.0000x reference)
import functools

import jax
import jax.numpy as jnp
from jax import lax
from jax.experimental import pallas as pl
from jax.experimental.pallas import tpu as pltpu
from jax.experimental.pallas import tpu_sc as plsc

N = 10000
E = 320000
D = 128
G = 16

NC = 2
NS = 16
NW = NC * NS
EPW = E // NW
CHUNK = 128
NFULL = EPW // CHUNK
TAIL = EPW - NFULL * CHUNK
ROWS_PER_TILE = N // NS
ZROWS = 125
CNT_STEPS = N // 16


def _sc_aggregate(count_deg):
    mesh = plsc.VectorSubcoreMesh(core_axis_name="c", subcore_axis_name="s")
    out_type = [jax.ShapeDtypeStruct((NC, N, D), jnp.float32)]
    scratch = [
        pltpu.VMEM((CHUNK,), jnp.int32),
        pltpu.VMEM((CHUNK,), jnp.int32),
        pltpu.VMEM((CHUNK, D), jnp.float32),
        pltpu.VMEM((ZROWS, D), jnp.float32),
        pltpu.VMEM_SHARED((N, D), jnp.float32),
        pltpu.SemaphoreType.DMA,
    ]
    if count_deg:
        out_type.append(jax.ShapeDtypeStruct((NW, N), jnp.float32))
        scratch.append(pltpu.VMEM((N,), jnp.float32))

    def body(h_hbm, src_hbm, dst_hbm, *rest):
        if count_deg:
            out_hbm, cnt_hbm, sidx, didx, rows, zbuf, acc, sem, cnt = rest
        else:
            out_hbm, sidx, didx, rows, zbuf, acc, sem = rest
        cid = lax.axis_index("c")
        sid = lax.axis_index("s")
        wid = cid * NS + sid

        zero16 = jnp.zeros((16,), jnp.float32)

        def zrow(r, carry):
            def zcol(c, carry2):
                zbuf[r, pl.ds(c * 16, 16)] = zero16
                return carry2
            return lax.fori_loop(0, D // 16, zcol, carry)
        lax.fori_loop(0, ZROWS, zrow, 0)

        row0 = sid * ROWS_PER_TILE

        def zacc(z, carry):
            pltpu.sync_copy(zbuf, acc.at[pl.ds(row0 + z * ZROWS, ZROWS)])
            return carry
        lax.fori_loop(0, ROWS_PER_TILE // ZROWS, zacc, 0)

        if count_deg:
            def zcnt(i, carry):
                cnt[pl.ds(i * 16, 16)] = zero16
                return carry
            lax.fori_loop(0, CNT_STEPS, zcnt, 0)

        plsc.subcore_barrier()

        ones16 = jnp.ones((16,), jnp.float32)
        ebase = wid * EPW

        def do_chunk(base, size):
            pltpu.sync_copy(src_hbm.at[pl.ds(base, size)], sidx.at[pl.ds(0, size)])
            pltpu.sync_copy(dst_hbm.at[pl.ds(base, size)], didx.at[pl.ds(0, size)])
            gidx = sidx if size == CHUNK else sidx.at[pl.ds(0, size)]
            widx = didx if size == CHUNK else didx.at[pl.ds(0, size)]
            grows = rows if size == CHUNK else rows.at[pl.ds(0, size)]
            pltpu.async_copy(h_hbm.at[gidx], grows, sem).wait()
            pltpu.sync_copy(grows, acc.at[widx], add=True)
            if count_deg:
                for i in range(size // 16):
                    idx16 = didx[pl.ds(i * 16, 16)]
                    plsc.addupdate_scatter(cnt, [idx16], ones16)

        def chunk_body(j, carry):
            do_chunk(ebase + j * CHUNK, CHUNK)
            return carry
        lax.fori_loop(0, NFULL, chunk_body, 0)
        if TAIL:
            do_chunk(ebase + NFULL * CHUNK, TAIL)

        if count_deg:
            pltpu.sync_copy(cnt, cnt_hbm.at[wid])

        plsc.subcore_barrier()

        pltpu.sync_copy(acc.at[pl.ds(row0, ROWS_PER_TILE)],
                        out_hbm.at[cid, pl.ds(row0, ROWS_PER_TILE)])

    return pl.kernel(
        body,
        mesh=mesh,
        out_type=tuple(out_type) if count_deg else out_type[0],
        scratch_types=scratch,
    )


BN = 1000
NBLK = N // BN


def _tc_layer_first(part, cnt, h, wl, wr, bias):
    ones32 = jnp.ones((NW, 1), jnp.float32)

    def body(part_ref, cnt_ref, h_ref, wl_ref, wr_ref, b_ref, out_ref, deg_ref):
        deg = lax.dot_general(cnt_ref[...], ones32, (((0,), (0,)), ((), ())),
                              preferred_element_type=jnp.float32)
        deg = jnp.maximum(deg, 1.0)
        deg_ref[...] = deg
        aggr = (part_ref[0] + part_ref[1]) * (1.0 / deg)
        out_ref[...] = (
            jnp.dot(aggr, wl_ref[...], preferred_element_type=jnp.float32)
            + jnp.dot(h_ref[...], wr_ref[...], preferred_element_type=jnp.float32)
            + b_ref[...]
        )

    return pl.pallas_call(
        body,
        grid=(NBLK,),
        in_specs=[
            pl.BlockSpec((NC, BN, D), lambda i: (0, i, 0)),
            pl.BlockSpec((NW, BN), lambda i: (0, i)),
            pl.BlockSpec((BN, D), lambda i: (i, 0)),
            pl.BlockSpec((D, D), lambda i: (0, 0)),
            pl.BlockSpec((D, D), lambda i: (0, 0)),
            pl.BlockSpec((1, D), lambda i: (0, 0)),
        ],
        out_specs=[
            pl.BlockSpec((BN, D), lambda i: (i, 0)),
            pl.BlockSpec((BN, 1), lambda i: (i, 0)),
        ],
        out_shape=[
            jax.ShapeDtypeStruct((N, D), jnp.float32),
            jax.ShapeDtypeStruct((N, 1), jnp.float32),
        ],
    )(part, cnt, h, wl, wr, bias)


def _tc_layer(part, deg, h, wl, wr, bias):
    def body(part_ref, deg_ref, h_ref, wl_ref, wr_ref, b_ref, out_ref):
        aggr = (part_ref[0] + part_ref[1]) * (1.0 / deg_ref[...])
        out_ref[...] = (
            jnp.dot(aggr, wl_ref[...], preferred_element_type=jnp.float32)
            + jnp.dot(h_ref[...], wr_ref[...], preferred_element_type=jnp.float32)
            + b_ref[...]
        )

    return pl.pallas_call(
        body,
        grid=(NBLK,),
        in_specs=[
            pl.BlockSpec((NC, BN, D), lambda i: (0, i, 0)),
            pl.BlockSpec((BN, 1), lambda i: (i, 0)),
            pl.BlockSpec((BN, D), lambda i: (i, 0)),
            pl.BlockSpec((D, D), lambda i: (0, 0)),
            pl.BlockSpec((D, D), lambda i: (0, 0)),
            pl.BlockSpec((1, D), lambda i: (0, 0)),
        ],
        out_specs=pl.BlockSpec((BN, D), lambda i: (i, 0)),
        out_shape=jax.ShapeDtypeStruct((N, D), jnp.float32),
    )(part, deg, h, wl, wr, bias)


def _tc_pool(h1, h2, h3, batch2, wp, bp):
    T = wp.shape[1]
    onesb = jnp.ones((BN, 1), jnp.float32)

    def body(h1_ref, h2_ref, h3_ref, b_ref, wp_ref, bp_ref, out_ref,
             acc_ref, cnt_ref):
        j = pl.program_id(0)

        @pl.when(j == 0)
        def _init():
            acc_ref[...] = jnp.zeros_like(acc_ref)
            cnt_ref[...] = jnp.zeros_like(cnt_ref)

        bt = b_ref[...]
        oh = (lax.broadcasted_iota(jnp.int32, (BN, G), 1) == bt).astype(jnp.float32)
        hcat = jnp.concatenate([h1_ref[...], h2_ref[...], h3_ref[...]], axis=1)
        acc_ref[...] += lax.dot_general(oh, hcat, (((0,), (0,)), ((), ())),
                                        preferred_element_type=jnp.float32)
        cnt_ref[...] += lax.dot_general(oh, onesb, (((0,), (0,)), ((), ())),
                                        preferred_element_type=jnp.float32)

        @pl.when(j == NBLK - 1)
        def _fin():
            pooled = acc_ref[...] / jnp.maximum(cnt_ref[...], 1.0)
            out_ref[...] = (
                jnp.dot(pooled, wp_ref[...], preferred_element_type=jnp.float32)
                + bp_ref[...]
            )

    return pl.pallas_call(
        body,
        grid=(NBLK,),
        in_specs=[
            pl.BlockSpec((BN, D), lambda i: (i, 0)),
            pl.BlockSpec((BN, D), lambda i: (i, 0)),
            pl.BlockSpec((BN, D), lambda i: (i, 0)),
            pl.BlockSpec((BN, 1), lambda i: (i, 0)),
            pl.BlockSpec((3 * D, T), lambda i: (0, 0)),
            pl.BlockSpec((1, T), lambda i: (0, 0)),
        ],
        out_specs=pl.BlockSpec((G, T), lambda i: (0, 0)),
        out_shape=jax.ShapeDtypeStruct((G, T), jnp.float32),
        scratch_shapes=[
            pltpu.VMEM((G, 3 * D), jnp.float32),
            pltpu.VMEM((G, 1), jnp.float32),
        ],
    )(h1, h2, h3, batch2, wp, bp)


_sc_aggr_first = _sc_aggregate(count_deg=True)
_sc_aggr = _sc_aggregate(count_deg=False)


def kernel(x, edge_index, batch, Wl, Wr, b, Wp, bp):
    src = edge_index[0]
    dst = edge_index[1]
    batch2 = batch.reshape(N, 1)

    part1, cnt = _sc_aggr_first(x, src, dst)
    h1, deg = _tc_layer_first(part1, cnt, x, Wl[0], Wr[0], b[0].reshape(1, D))
    part2 = _sc_aggr(h1, src, dst)
    h2 = _tc_layer(part2, deg, h1, Wl[1], Wr[1], b[1].reshape(1, D))
    part3 = _sc_aggr(h2, src, dst)
    h3 = _tc_layer(part3, deg, h2, Wl[2], Wr[2], b[2].reshape(1, D))
    return _tc_pool(h1, h2, h3, batch2, Wp, bp.reshape(1, bp.shape[0]))


# speedup vs baseline: 4.0714x; 4.0714x over previous
import functools

import jax
import jax.numpy as jnp
from jax import lax
from jax.experimental import pallas as pl
from jax.experimental.pallas import tpu as pltpu
from jax.experimental.pallas import tpu_sc as plsc

N = 10000
E = 320000
D = 128
DH = D // 2
G = 16

NC = 2
NS = 16
NW = NC * NS
EPT = E // NS
CHUNK = 128
NFULL = EPT // CHUNK
TAIL = EPT - NFULL * CHUNK
CROWS = 632
CROWS_LAST = N - 15 * CROWS


def _sc_aggregate(count_deg):
    mesh = plsc.VectorSubcoreMesh(core_axis_name="c", subcore_axis_name="s")
    out_type = [jax.ShapeDtypeStruct((NC, N, DH), jnp.float32)]
    scratch = [
        pltpu.VMEM((CHUNK,), jnp.int32),
        pltpu.VMEM((CHUNK,), jnp.int32),
        pltpu.VMEM((CHUNK, DH), jnp.float32),
        pltpu.VMEM((CROWS, DH), jnp.float32),
        pltpu.VMEM_SHARED((N, DH), jnp.float32),
        pltpu.SemaphoreType.DMA,
    ]
    if count_deg:
        out_type.append(jax.ShapeDtypeStruct((NW, 1, N), jnp.float32))
        scratch.append(pltpu.VMEM((N,), jnp.float32))

    def body(h0_hbm, h1_hbm, src_hbm, dst_hbm, *rest):
        if count_deg:
            out_hbm, cnt_hbm, sidx, didx, rows, zbuf, acc, sem, cnt = rest
        else:
            out_hbm, sidx, didx, rows, zbuf, acc, sem = rest
        cid = lax.axis_index("c")
        sid = lax.axis_index("s")
        wid = cid * NS + sid

        zero16 = jnp.zeros((16,), jnp.float32)
        ones16 = jnp.ones((16,), jnp.float32)

        def zrow(r, carry):
            def zcol(c, carry2):
                zbuf[r, pl.ds(c * 16, 16)] = zero16
                return carry2
            return lax.fori_loop(0, DH // 16, zcol, carry)
        lax.fori_loop(0, CROWS, zrow, 0)

        row0 = sid * CROWS

        @pl.when(sid < NS - 1)
        def _zfull():
            pltpu.sync_copy(zbuf, acc.at[pl.ds(row0, CROWS)])

        @pl.when(sid == NS - 1)
        def _zlast():
            pltpu.sync_copy(zbuf.at[pl.ds(0, CROWS_LAST)],
                            acc.at[pl.ds((NS - 1) * CROWS, CROWS_LAST)])

        if count_deg:
            def zcnt(i, carry):
                cnt[pl.ds(i * 16, 16)] = zero16
                return carry
            lax.fori_loop(0, N // 16, zcnt, 0)

        plsc.subcore_barrier()

        ebase = sid * EPT

        def do_chunk(base, size):
            pltpu.sync_copy(src_hbm.at[pl.ds(base, size)], sidx.at[pl.ds(0, size)])
            pltpu.sync_copy(dst_hbm.at[pl.ds(base, size)], didx.at[pl.ds(0, size)])
            gidx = sidx if size == CHUNK else sidx.at[pl.ds(0, size)]
            widx = didx if size == CHUNK else didx.at[pl.ds(0, size)]
            grows = rows if size == CHUNK else rows.at[pl.ds(0, size)]

            @pl.when(cid == 0)
            def _g0():
                pltpu.async_copy(h0_hbm.at[gidx], grows, sem).wait()

            @pl.when(cid == 1)
            def _g1():
                pltpu.async_copy(h1_hbm.at[gidx], grows, sem).wait()

            pltpu.sync_copy(grows, acc.at[widx], add=True)
            if count_deg:
                for i in range(size // 16):
                    idx16 = didx[pl.ds(i * 16, 16)]
                    plsc.addupdate_scatter(cnt, [idx16], ones16)

        def chunk_body(j, carry):
            do_chunk(ebase + j * CHUNK, CHUNK)
            return carry
        lax.fori_loop(0, NFULL, chunk_body, 0)
        if TAIL:
            do_chunk(ebase + NFULL * CHUNK, TAIL)

        if count_deg:
            pltpu.sync_copy(cnt, cnt_hbm.at[wid, 0])

        plsc.subcore_barrier()

        @pl.when(sid < NS - 1)
        def _cfull():
            pltpu.sync_copy(acc.at[pl.ds(row0, CROWS)],
                            out_hbm.at[cid, pl.ds(row0, CROWS)])

        @pl.when(sid == NS - 1)
        def _clast():
            pltpu.sync_copy(acc.at[pl.ds((NS - 1) * CROWS, CROWS_LAST)],
                            out_hbm.at[cid, pl.ds((NS - 1) * CROWS, CROWS_LAST)])

    return pl.kernel(
        body,
        mesh=mesh,
        out_type=tuple(out_type) if count_deg else out_type[0],
        scratch_types=scratch,
        compiler_params=pltpu.CompilerParams(
            needs_layout_passes=False, use_tc_tiling_on_sc=False),
    )


BN = 1000
NBLK = N // BN


def _tc_layer_first(aggr, cnt_t, h, wl, wr, bias):
    def body(a_ref, cnt_ref, h_ref, wl_ref, wr_ref, b_ref,
             out_ref, out2_ref, deg_ref):
        ones32 = jnp.ones((NW, 1), jnp.float32)
        deg = jnp.dot(cnt_ref[...], ones32,
                      preferred_element_type=jnp.float32)
        deg = jnp.maximum(deg * 0.5, 1.0)
        deg_ref[...] = deg
        aggr_c = jnp.concatenate([a_ref[0], a_ref[1]], axis=1) * (1.0 / deg)
        res = (
            jnp.dot(aggr_c, wl_ref[...], preferred_element_type=jnp.float32)
            + jnp.dot(h_ref[...], wr_ref[...], preferred_element_type=jnp.float32)
            + b_ref[...]
        )
        out_ref[...] = res[:, :DH]
        out2_ref[...] = res[:, DH:]

    return pl.pallas_call(
        body,
        grid=(NBLK,),
        in_specs=[
            pl.BlockSpec((NC, BN, DH), lambda i: (0, i, 0)),
            pl.BlockSpec((BN, NW), lambda i: (i, 0)),
            pl.BlockSpec((BN, D), lambda i: (i, 0)),
            pl.BlockSpec((D, D), lambda i: (0, 0)),
            pl.BlockSpec((D, D), lambda i: (0, 0)),
            pl.BlockSpec((1, D), lambda i: (0, 0)),
        ],
        out_specs=[
            pl.BlockSpec((BN, DH), lambda i: (i, 0)),
            pl.BlockSpec((BN, DH), lambda i: (i, 0)),
            pl.BlockSpec((BN, 1), lambda i: (i, 0)),
        ],
        out_shape=[
            jax.ShapeDtypeStruct((N, DH), jnp.float32),
            jax.ShapeDtypeStruct((N, DH), jnp.float32),
            jax.ShapeDtypeStruct((N, 1), jnp.float32),
        ],
    )(aggr, cnt_t, h, wl, wr, bias)


def _tc_layer(aggr, deg, h0, h1, wl, wr, bias):
    def body(a_ref, deg_ref, h0_ref, h1_ref, wl_ref, wr_ref, b_ref,
             out_ref, out2_ref):
        aggr_c = jnp.concatenate([a_ref[0], a_ref[1]], axis=1) / deg_ref[...]
        h_c = jnp.concatenate([h0_ref[...], h1_ref[...]], axis=1)
        res = (
            jnp.dot(aggr_c, wl_ref[...], preferred_element_type=jnp.float32)
            + jnp.dot(h_c, wr_ref[...], preferred_element_type=jnp.float32)
            + b_ref[...]
        )
        out_ref[...] = res[:, :DH]
        out2_ref[...] = res[:, DH:]

    return pl.pallas_call(
        body,
        grid=(NBLK,),
        in_specs=[
            pl.BlockSpec((NC, BN, DH), lambda i: (0, i, 0)),
            pl.BlockSpec((BN, 1), lambda i: (i, 0)),
            pl.BlockSpec((BN, DH), lambda i: (i, 0)),
            pl.BlockSpec((BN, DH), lambda i: (i, 0)),
            pl.BlockSpec((D, D), lambda i: (0, 0)),
            pl.BlockSpec((D, D), lambda i: (0, 0)),
            pl.BlockSpec((1, D), lambda i: (0, 0)),
        ],
        out_specs=[
            pl.BlockSpec((BN, DH), lambda i: (i, 0)),
            pl.BlockSpec((BN, DH), lambda i: (i, 0)),
        ],
        out_shape=[
            jax.ShapeDtypeStruct((N, DH), jnp.float32),
            jax.ShapeDtypeStruct((N, DH), jnp.float32),
        ],
    )(aggr, deg, h0, h1, wl, wr, bias)


def _tc_pool(hs, batch2, wp, bp):
    T = wp.shape[1]

    def body(h10_ref, h11_ref, h20_ref, h21_ref, h30_ref, h31_ref,
             b_ref, wp_ref, bp_ref, out_ref, acc_ref, cnt_ref):
        j = pl.program_id(0)

        @pl.when(j == 0)
        def _init():
            acc_ref[...] = jnp.zeros_like(acc_ref)
            cnt_ref[...] = jnp.zeros_like(cnt_ref)

        bt = b_ref[...]
        oh = (lax.broadcasted_iota(jnp.int32, (BN, G), 1) == bt).astype(jnp.float32)
        hcat = jnp.concatenate(
            [h10_ref[...], h11_ref[...], h20_ref[...], h21_ref[...],
             h30_ref[...], h31_ref[...]], axis=1)
        acc_ref[...] += lax.dot_general(oh, hcat, (((0,), (0,)), ((), ())),
                                        preferred_element_type=jnp.float32)
        onesb = jnp.ones((BN, 1), jnp.float32)
        cnt_ref[...] += lax.dot_general(oh, onesb, (((0,), (0,)), ((), ())),
                                        preferred_element_type=jnp.float32)

        @pl.when(j == NBLK - 1)
        def _fin():
            pooled = acc_ref[...] / jnp.maximum(cnt_ref[...], 1.0)
            out_ref[...] = (
                jnp.dot(pooled, wp_ref[...], preferred_element_type=jnp.float32)
                + bp_ref[...]
            )

    return pl.pallas_call(
        body,
        grid=(NBLK,),
        in_specs=(
            [pl.BlockSpec((BN, DH), lambda i: (i, 0))] * 6
            + [
                pl.BlockSpec((BN, 1), lambda i: (i, 0)),
                pl.BlockSpec((3 * D, T), lambda i: (0, 0)),
                pl.BlockSpec((1, T), lambda i: (0, 0)),
            ]
        ),
        out_specs=pl.BlockSpec((G, T), lambda i: (0, 0)),
        out_shape=jax.ShapeDtypeStruct((G, T), jnp.float32),
        scratch_shapes=[
            pltpu.VMEM((G, 3 * D), jnp.float32),
            pltpu.VMEM((G, 1), jnp.float32),
        ],
    )(*hs, batch2, wp, bp)


_sc_aggr_first = _sc_aggregate(count_deg=True)
_sc_aggr = _sc_aggregate(count_deg=False)


def kernel(x, edge_index, batch, Wl, Wr, b, Wp, bp):
    src = edge_index[0]
    dst = edge_index[1]
    batch2 = batch.reshape(N, 1)
    x0 = x[:, :DH]
    x1 = x[:, DH:]

    aggr1, cnt = _sc_aggr_first(x0, x1, src, dst)
    cnt_t = cnt.reshape(NW, N).T
    h10, h11, deg = _tc_layer_first(aggr1, cnt_t, x,
                                    Wl[0], Wr[0], b[0].reshape(1, D))
    aggr2 = _sc_aggr(h10, h11, src, dst)
    h20, h21 = _tc_layer(aggr2, deg, h10, h11, Wl[1], Wr[1], b[1].reshape(1, D))
    aggr3 = _sc_aggr(h20, h21, src, dst)
    h30, h31 = _tc_layer(aggr3, deg, h20, h21, Wl[2], Wr[2], b[2].reshape(1, D))

    return _tc_pool((h10, h11, h20, h21, h30, h31), batch2,
                    Wp, bp.reshape(1, bp.shape[0]))
